# Initial kernel scaffold; baseline (speedup 1.0000x reference)
import jax
import jax.numpy as jnp
from jax.experimental import pallas as pl


def kernel(passage, question, embedding, p_Wx_f, p_Wh_f, p_b_f, p_Wx_b, p_Wh_b, p_b_b, q_Wx_f, q_Wh_f, q_b_f, q_Wx_b, q_Wh_b, q_b_b, W_att, W_m, b_m, w_start, w_end):
    raise NotImplementedError("write your pallas kernel here")


# speedup vs baseline: 4.4620x; 4.4620x over previous
import functools

import jax
import jax.numpy as jnp
from jax import lax
from jax.experimental import pallas as pl
from jax.experimental.pallas import tpu as pltpu
from jax.experimental.pallas import tpu_sc as plsc

B = 16
P = 400
Q = 30
V = 100000
D = 128
H = 256

NP_TOK = B * P
NQ_TOK = B * Q
Q_PAD = 512
NTOK = NP_TOK + Q_PAD

_INTERPRET = False

try:
    _info = plsc.get_sparse_core_info()
    _NUM_WORKERS = _info.num_cores * _info.num_subcores
except Exception:
    _NUM_WORKERS = 32


def _gather_rows(table, idx):
    n_workers = _NUM_WORKERS
    per_w = NTOK // n_workers
    chunk = 72
    n_ch = per_w // chunk

    mesh = plsc.VectorSubcoreMesh(core_axis_name="c", subcore_axis_name="s")

    @functools.partial(
        pl.kernel,
        mesh=mesh,
        out_type=jax.ShapeDtypeStruct((NTOK, D), jnp.float32),
        scratch_types=[
            pltpu.VMEM((n_ch, chunk), jnp.int32),
            pltpu.VMEM((per_w, D), jnp.float32),
            pltpu.SemaphoreType.DMA,
        ],
    )
    def gk(table_hbm, idx_hbm, out_hbm, idx_v, rows_v, sem):
        n_cores = n_workers // 16
        wid = lax.axis_index("s") * n_cores + lax.axis_index("c")
        base = wid * per_w
        for j in range(n_ch):
            pltpu.sync_copy(idx_hbm.at[pl.ds(base + j * chunk, chunk)], idx_v.at[j])
        copies = []
        for j in range(n_ch):
            copies.append(
                pltpu.async_copy(
                    table_hbm.at[idx_v.at[j]],
                    rows_v.at[pl.ds(j * chunk, chunk)],
                    sem,
                )
            )
        for c in copies:
            c.wait()
        pltpu.sync_copy(rows_v, out_hbm.at[pl.ds(base, per_w)])

    return gk(table, idx)


def _dot(a, b):
    return lax.dot_general(
        a, b, (((1,), (0,)), ((), ())),
        precision=lax.Precision.HIGHEST,
        preferred_element_type=jnp.float32,
    )


def _xw_project(rows, w_stack, b_stack):
    MB = 256
    n_blk = NTOK // MB
    n_p = NP_TOK // MB

    def body(x_ref, w_ref, b_ref, o_ref):
        o_ref[...] = _dot(x_ref[...], w_ref[0]) + b_ref[0]

    return pl.pallas_call(
        body,
        grid=(n_blk,),
        in_specs=[
            pl.BlockSpec((MB, D), lambda i: (i, 0)),
            pl.BlockSpec((1, D, 8 * H), lambda i: (jnp.where(i < n_p, 0, 1), 0, 0)),
            pl.BlockSpec((1, 1, 8 * H), lambda i: (jnp.where(i < n_p, 0, 1), 0, 0)),
        ],
        out_specs=pl.BlockSpec((MB, 8 * H), lambda i: (i, 0)),
        out_shape=jax.ShapeDtypeStruct((NTOK, 8 * H), jnp.float32),
        interpret=_INTERPRET,
    )(rows, w_stack, b_stack)


def _lstm_step(x, h, c, wh):
    z = x + _dot(h, wh)
    i = jax.nn.sigmoid(z[:, 0:H])
    f = jax.nn.sigmoid(z[:, H:2 * H])
    g = jnp.tanh(z[:, 2 * H:3 * H])
    o = jax.nn.sigmoid(z[:, 3 * H:4 * H])
    c = f * c + i * g
    h = o * jnp.tanh(c)
    return h, c


def _bilstm_scan(xw, wh_f, wh_b, T, CH):
    n_ch = T // CH

    def body(xf_ref, xb_ref, whf_ref, whb_ref, hf_o, hb_o, hf_s, cf_s, hb_s, cb_s):
        i = pl.program_id(0)

        @pl.when(i == 0)
        def _():
            z = jnp.zeros((B, H), jnp.float32)
            hf_s[...] = z
            cf_s[...] = z
            hb_s[...] = z
            cb_s[...] = z

        hf, cf, hb, cb = hf_s[...], cf_s[...], hb_s[...], cb_s[...]
        whf = whf_ref[...]
        whb = whb_ref[...]
        for tl in range(CH):
            hf, cf = _lstm_step(xf_ref[tl], hf, cf, whf)
            hf_o[:, tl, :] = hf
            hb, cb = _lstm_step(xb_ref[CH - 1 - tl], hb, cb, whb)
            hb_o[:, CH - 1 - tl, :] = hb
        hf_s[...] = hf
        cf_s[...] = cf
        hb_s[...] = hb
        cb_s[...] = cb

    return pl.pallas_call(
        body,
        grid=(n_ch,),
        in_specs=[
            pl.BlockSpec((CH, B, 4 * H), lambda i: (i, 0, 0)),
            pl.BlockSpec((CH, B, 4 * H), lambda i: (n_ch - 1 - i, 0, 1)),
            pl.BlockSpec((H, 4 * H), lambda i: (0, 0)),
            pl.BlockSpec((H, 4 * H), lambda i: (0, 0)),
        ],
        out_specs=[
            pl.BlockSpec((B, CH, H), lambda i: (0, i, 0)),
            pl.BlockSpec((B, CH, H), lambda i: (0, n_ch - 1 - i, 0)),
        ],
        out_shape=[
            jax.ShapeDtypeStruct((B, T, H), jnp.float32),
            jax.ShapeDtypeStruct((B, T, H), jnp.float32),
        ],
        scratch_shapes=[pltpu.VMEM((B, H), jnp.float32)] * 4,
        interpret=_INTERPRET,
    )(xw, xw, wh_f, wh_b)


def _attention(hpf, hpb, hqf, hqb, w_att, w_m, b_m2, w_se):
    def body(hpf_r, hpb_r, hqf_r, hqb_r, wa_r, wm_r, bm_r, wse_r, lo_r, pr_r):
        hp = jnp.concatenate([hpf_r[0], hpb_r[0]], axis=-1)
        hq = jnp.concatenate([hqf_r[0], hqb_r[0]], axis=-1)
        sp = _dot(hp, wa_r[...])
        scores = lax.dot_general(
            sp, hq, (((1,), (1,)), ((), ())),
            precision=lax.Precision.HIGHEST,
            preferred_element_type=jnp.float32,
        )
        mx = jnp.max(scores, axis=-1, keepdims=True)
        e = jnp.exp(scores - mx)
        alpha = e / jnp.sum(e, axis=-1, keepdims=True)
        ctx = _dot(alpha, hq)
        pre = _dot(hp, wm_r[0:2 * H, :]) + _dot(ctx, wm_r[2 * H:4 * H, :]) + bm_r[...]
        m = jnp.tanh(pre)
        lt = lax.dot_general(
            wse_r[...], m, (((0,), (1,)), ((), ())),
            precision=lax.Precision.HIGHEST,
            preferred_element_type=jnp.float32,
        )
        lo_r[0] = lt
        iota = lax.broadcasted_iota(jnp.int32, (2, P), 1)
        mx2 = jnp.max(lt, axis=-1, keepdims=True)
        idx = jnp.min(jnp.where(lt == mx2, iota, P), axis=-1)
        pr_r[0] = idx.reshape(1, 2)

    return pl.pallas_call(
        body,
        grid=(B,),
        in_specs=[
            pl.BlockSpec((1, P, H), lambda b: (b, 0, 0)),
            pl.BlockSpec((1, P, H), lambda b: (b, 0, 0)),
            pl.BlockSpec((1, Q, H), lambda b: (b, 0, 0)),
            pl.BlockSpec((1, Q, H), lambda b: (b, 0, 0)),
            pl.BlockSpec((2 * H, 2 * H), lambda b: (0, 0)),
            pl.BlockSpec((4 * H, 2 * H), lambda b: (0, 0)),
            pl.BlockSpec((1, 2 * H), lambda b: (0, 0)),
            pl.BlockSpec((2 * H, 2), lambda b: (0, 0)),
        ],
        out_specs=[
            pl.BlockSpec((1, 2, P), lambda b: (b, 0, 0)),
            pl.BlockSpec((1, 1, 2), lambda b: (b, 0, 0)),
        ],
        out_shape=[
            jax.ShapeDtypeStruct((B, 2, P), jnp.float32),
            jax.ShapeDtypeStruct((B, 1, 2), jnp.int32),
        ],
        interpret=_INTERPRET,
    )(hpf, hpb, hqf, hqb, w_att, w_m, b_m2, w_se)


def kernel(passage, question, embedding,
           p_Wx_f, p_Wh_f, p_b_f, p_Wx_b, p_Wh_b, p_b_b,
           q_Wx_f, q_Wh_f, q_b_f, q_Wx_b, q_Wh_b, q_b_b,
           W_att, W_m, b_m, w_start, w_end):
    pidx = jnp.transpose(passage).reshape(-1).astype(jnp.int32)
    qidx = jnp.transpose(question[:, ::-1]).reshape(-1).astype(jnp.int32)
    idx = jnp.concatenate([pidx, qidx, jnp.zeros((Q_PAD - NQ_TOK,), jnp.int32)])

    rows = _gather_rows(embedding, idx)

    w_stack = jnp.stack([
        jnp.concatenate([p_Wx_f, p_Wx_b], axis=1),
        jnp.concatenate([q_Wx_f, q_Wx_b], axis=1),
    ])
    b_stack = jnp.stack([
        jnp.concatenate([p_b_f, p_b_b]),
        jnp.concatenate([q_b_f, q_b_b]),
    ]).reshape(2, 1, 8 * H)

    xw_all = _xw_project(rows, w_stack, b_stack)
    xw_p = xw_all[:NP_TOK].reshape(P, B, 8 * H)
    xw_q = xw_all[NP_TOK:NP_TOK + NQ_TOK].reshape(Q, B, 8 * H)

    hpf, hpb = _bilstm_scan(xw_p, p_Wh_f, p_Wh_b, P, 40)
    hqf, hqb = _bilstm_scan(xw_q, q_Wh_f, q_Wh_b, Q, 30)

    logits, preds = _attention(
        hpf, hpb, hqf, hqb, W_att, W_m,
        b_m.reshape(1, 2 * H), jnp.stack([w_start, w_end], axis=1),
    )
    return logits, preds.reshape(B, 2)


# speedup vs baseline: 13.5411x; 3.0348x over previous
import functools

import jax
import jax.numpy as jnp
from jax import lax
from jax.experimental import pallas as pl
from jax.experimental.pallas import tpu as pltpu
from jax.experimental.pallas import tpu_sc as plsc

B = 16
P = 400
Q = 30
V = 100000
D = 128
H = 256

NP_TOK = B * P
NQ_TOK = B * Q
Q_PAD = 512
NTOK = NP_TOK + Q_PAD

_INTERPRET = False

try:
    _info = plsc.get_sparse_core_info()
    _NUM_WORKERS = _info.num_cores * _info.num_subcores
except Exception:
    _NUM_WORKERS = 32


def _gather_rows(table, idx):
    n_workers = _NUM_WORKERS
    per_w = NTOK // n_workers
    chunk = 72
    n_ch = per_w // chunk

    mesh = plsc.VectorSubcoreMesh(core_axis_name="c", subcore_axis_name="s")

    @functools.partial(
        pl.kernel,
        mesh=mesh,
        out_type=jax.ShapeDtypeStruct((NTOK, D), jnp.float32),
        scratch_types=[
            pltpu.VMEM((n_ch, chunk), jnp.int32),
            pltpu.VMEM((per_w, D), jnp.float32),
            pltpu.SemaphoreType.DMA,
        ],
    )
    def gk(table_hbm, idx_hbm, out_hbm, idx_v, rows_v, sem):
        n_cores = n_workers // 16
        wid = lax.axis_index("s") * n_cores + lax.axis_index("c")
        base = wid * per_w
        for j in range(n_ch):
            pltpu.sync_copy(idx_hbm.at[pl.ds(base + j * chunk, chunk)], idx_v.at[j])
        copies = []
        for j in range(n_ch):
            copies.append(
                pltpu.async_copy(
                    table_hbm.at[idx_v.at[j]],
                    rows_v.at[pl.ds(j * chunk, chunk)],
                    sem,
                )
            )
        for c in copies:
            c.wait()
        pltpu.sync_copy(rows_v, out_hbm.at[pl.ds(base, per_w)])

    return gk(table, idx)


def _dot(a, b):
    return lax.dot_general(
        a, b, (((1,), (0,)), ((), ())),
        preferred_element_type=jnp.float32,
    )


def _dot_t(a, b):
    return lax.dot_general(
        a, b, (((1,), (1,)), ((), ())),
        preferred_element_type=jnp.float32,
    )


def _xw_project(rows, w_stack):
    MB = 256
    n_blk = NTOK // MB
    n_p = NP_TOK // MB

    def body(x_ref, w_ref, o_ref):
        o_ref[...] = _dot(x_ref[...], w_ref[0])

    return pl.pallas_call(
        body,
        grid=(n_blk,),
        in_specs=[
            pl.BlockSpec((MB, D), lambda i: (i, 0)),
            pl.BlockSpec((1, D, 8 * H), lambda i: (jnp.where(i < n_p, 0, 1), 0, 0)),
        ],
        out_specs=pl.BlockSpec((MB, 8 * H), lambda i: (i, 0)),
        out_shape=jax.ShapeDtypeStruct((NTOK, 8 * H), jnp.float32),
        interpret=_INTERPRET,
    )(rows, w_stack)


def _lstm_step(x, h, c, wh, b):
    z = (x + _dot(h, wh)) + b
    i = jax.nn.sigmoid(z[:, 0:H])
    f = jax.nn.sigmoid(z[:, H:2 * H])
    g = jnp.tanh(z[:, 2 * H:3 * H])
    o = jax.nn.sigmoid(z[:, 3 * H:4 * H])
    c = f * c + i * g
    h = o * jnp.tanh(c)
    return h, c


def _bilstm_scan(xw, wh_f, wh_b, b_f, b_b, T, CH):
    n_ch = T // CH

    def body(xf_ref, xb_ref, whf_ref, whb_ref, bf_ref, bb_ref, hf_o, hb_o,
             hf_s, cf_s, hb_s, cb_s):
        i = pl.program_id(0)

        @pl.when(i == 0)
        def _():
            z = jnp.zeros((B, H), jnp.float32)
            hf_s[...] = z
            cf_s[...] = z
            hb_s[...] = z
            cb_s[...] = z

        hf, cf, hb, cb = hf_s[...], cf_s[...], hb_s[...], cb_s[...]
        whf = whf_ref[...]
        whb = whb_ref[...]
        bf = bf_ref[...]
        bb = bb_ref[...]
        for tl in range(CH):
            hf, cf = _lstm_step(xf_ref[tl], hf, cf, whf, bf)
            hf_o[:, tl, :] = hf
            hb, cb = _lstm_step(xb_ref[CH - 1 - tl], hb, cb, whb, bb)
            hb_o[:, CH - 1 - tl, :] = hb
        hf_s[...] = hf
        cf_s[...] = cf
        hb_s[...] = hb
        cb_s[...] = cb

    return pl.pallas_call(
        body,
        grid=(n_ch,),
        in_specs=[
            pl.BlockSpec((CH, B, 4 * H), lambda i: (i, 0, 0)),
            pl.BlockSpec((CH, B, 4 * H), lambda i: (n_ch - 1 - i, 0, 1)),
            pl.BlockSpec((H, 4 * H), lambda i: (0, 0)),
            pl.BlockSpec((H, 4 * H), lambda i: (0, 0)),
            pl.BlockSpec((1, 4 * H), lambda i: (0, 0)),
            pl.BlockSpec((1, 4 * H), lambda i: (0, 0)),
        ],
        out_specs=[
            pl.BlockSpec((B, CH, H), lambda i: (0, i, 0)),
            pl.BlockSpec((B, CH, H), lambda i: (0, n_ch - 1 - i, 0)),
        ],
        out_shape=[
            jax.ShapeDtypeStruct((B, T, H), jnp.float32),
            jax.ShapeDtypeStruct((B, T, H), jnp.float32),
        ],
        scratch_shapes=[pltpu.VMEM((B, H), jnp.float32)] * 4,
        interpret=_INTERPRET,
    )(xw, xw, wh_f, wh_b, b_f, b_b)


def _attention(hpf, hpb, hqf, hqb, w_att, w_m, b_m2, w_se):
    def body(hpf_r, hpb_r, hqf_r, hqb_r, wa_r, wm_r, bm_r, wse_r, lo_r, pr_r):
        hp = jnp.concatenate([hpf_r[0], hpb_r[0]], axis=-1)
        hq = jnp.concatenate([hqf_r[0], hqb_r[0]], axis=-1)
        tmp = _dot_t(hq, wa_r[...])
        scores = _dot_t(hp, tmp)
        mx = jnp.max(scores, axis=-1, keepdims=True)
        e = jnp.exp(scores - mx)
        alpha = e / jnp.sum(e, axis=-1, keepdims=True)
        ctx = _dot(alpha, hq)
        cat = jnp.concatenate([hp, ctx], axis=-1)
        m = jnp.tanh(_dot(cat, wm_r[...]) + bm_r[...])
        lt = lax.dot_general(
            wse_r[...], m, (((0,), (1,)), ((), ())),
            preferred_element_type=jnp.float32,
        )
        lo_r[0] = lt
        iota = lax.broadcasted_iota(jnp.int32, (2, P), 1)
        mx2 = jnp.max(lt, axis=-1, keepdims=True)
        idx = jnp.min(jnp.where(lt == mx2, iota, P), axis=-1)
        pr_r[0] = idx.reshape(1, 2)

    return pl.pallas_call(
        body,
        grid=(B,),
        in_specs=[
            pl.BlockSpec((1, P, H), lambda b: (b, 0, 0)),
            pl.BlockSpec((1, P, H), lambda b: (b, 0, 0)),
            pl.BlockSpec((1, Q, H), lambda b: (b, 0, 0)),
            pl.BlockSpec((1, Q, H), lambda b: (b, 0, 0)),
            pl.BlockSpec((2 * H, 2 * H), lambda b: (0, 0)),
            pl.BlockSpec((4 * H, 2 * H), lambda b: (0, 0)),
            pl.BlockSpec((1, 2 * H), lambda b: (0, 0)),
            pl.BlockSpec((2 * H, 2), lambda b: (0, 0)),
        ],
        out_specs=[
            pl.BlockSpec((1, 2, P), lambda b: (b, 0, 0)),
            pl.BlockSpec((1, 1, 2), lambda b: (b, 0, 0)),
        ],
        out_shape=[
            jax.ShapeDtypeStruct((B, 2, P), jnp.float32),
            jax.ShapeDtypeStruct((B, 1, 2), jnp.int32),
        ],
        interpret=_INTERPRET,
    )(hpf, hpb, hqf, hqb, w_att, w_m, b_m2, w_se)


def kernel(passage, question, embedding,
           p_Wx_f, p_Wh_f, p_b_f, p_Wx_b, p_Wh_b, p_b_b,
           q_Wx_f, q_Wh_f, q_b_f, q_Wx_b, q_Wh_b, q_b_b,
           W_att, W_m, b_m, w_start, w_end):
    pidx = jnp.transpose(passage).reshape(-1).astype(jnp.int32)
    qidx = jnp.transpose(question[:, ::-1]).reshape(-1).astype(jnp.int32)
    idx = jnp.concatenate([pidx, qidx, jnp.zeros((Q_PAD - NQ_TOK,), jnp.int32)])

    rows = _gather_rows(embedding, idx)

    w_stack = jnp.stack([
        jnp.concatenate([p_Wx_f, p_Wx_b], axis=1),
        jnp.concatenate([q_Wx_f, q_Wx_b], axis=1),
    ])

    xw_all = _xw_project(rows, w_stack)
    xw_p = xw_all[:NP_TOK].reshape(P, B, 8 * H)
    xw_q = xw_all[NP_TOK:NP_TOK + NQ_TOK].reshape(Q, B, 8 * H)

    hpf, hpb = _bilstm_scan(xw_p, p_Wh_f, p_Wh_b,
                            p_b_f.reshape(1, 4 * H), p_b_b.reshape(1, 4 * H), P, 40)
    hqf, hqb = _bilstm_scan(xw_q, q_Wh_f, q_Wh_b,
                            q_b_f.reshape(1, 4 * H), q_b_b.reshape(1, 4 * H), Q, 30)

    logits, preds = _attention(
        hpf, hpb, hqf, hqb, W_att, W_m,
        b_m.reshape(1, 2 * H), jnp.stack([w_start, w_end], axis=1),
    )
    return logits, preds.reshape(B, 2)


# speedup vs baseline: 18.6412x; 1.3766x over previous
import functools

import jax
import jax.numpy as jnp
from jax import lax
from jax.experimental import pallas as pl
from jax.experimental.pallas import tpu as pltpu
from jax.experimental.pallas import tpu_sc as plsc

B = 16
P = 400
Q = 30
V = 100000
D = 128
H = 256

NP_TOK = B * P
NQ_TOK = B * Q
Q_PAD = 512
NTOK = NP_TOK + Q_PAD

_INTERPRET = False

try:
    _info = plsc.get_sparse_core_info()
    _NUM_WORKERS = _info.num_cores * _info.num_subcores
except Exception:
    _NUM_WORKERS = 32


def _gather_rows(table, idx):
    n_workers = _NUM_WORKERS
    per_w = NTOK // n_workers
    chunk = 72
    n_ch = per_w // chunk

    mesh = plsc.VectorSubcoreMesh(core_axis_name="c", subcore_axis_name="s")

    @functools.partial(
        pl.kernel,
        mesh=mesh,
        out_type=jax.ShapeDtypeStruct((NTOK, D), jnp.float32),
        scratch_types=[
            pltpu.VMEM((n_ch, chunk), jnp.int32),
            pltpu.VMEM((per_w, D), jnp.float32),
            pltpu.SemaphoreType.DMA,
        ],
    )
    def gk(table_hbm, idx_hbm, out_hbm, idx_v, rows_v, sem):
        n_cores = n_workers // 16
        wid = lax.axis_index("s") * n_cores + lax.axis_index("c")
        base = wid * per_w
        for j in range(n_ch):
            pltpu.sync_copy(idx_hbm.at[pl.ds(base + j * chunk, chunk)], idx_v.at[j])
        copies = []
        for j in range(n_ch):
            copies.append(
                pltpu.async_copy(
                    table_hbm.at[idx_v.at[j]],
                    rows_v.at[pl.ds(j * chunk, chunk)],
                    sem,
                )
            )
        for c in copies:
            c.wait()
        pltpu.sync_copy(rows_v, out_hbm.at[pl.ds(base, per_w)])

    return gk(table, idx)


def _dot(a, b):
    return lax.dot_general(
        a, b, (((1,), (0,)), ((), ())),
        preferred_element_type=jnp.float32,
    )


def _dot_t(a, b):
    return lax.dot_general(
        a, b, (((1,), (1,)), ((), ())),
        preferred_element_type=jnp.float32,
    )


def _lstm_step(x, h, c, wh, b):
    z = (x + _dot(h, wh)) + b
    i = jax.nn.sigmoid(z[:, 0:H])
    f = jax.nn.sigmoid(z[:, H:2 * H])
    g = jnp.tanh(z[:, 2 * H:3 * H])
    o = jax.nn.sigmoid(z[:, 3 * H:4 * H])
    c = f * c + i * g
    h = o * jnp.tanh(c)
    return h, c


def _bilstm_scan(rows, wx_f, wx_b, wh_f, wh_b, b_f, b_b, T, CH):
    n_ch = T // CH

    def body(xf_ref, xb_ref, wxf_ref, wxb_ref, whf_ref, whb_ref, bf_ref, bb_ref,
             hf_o, hb_o, xwf_s, xwb_s, hf_s, cf_s, hb_s, cb_s):
        i = pl.program_id(0)

        @pl.when(i == 0)
        def _():
            z = jnp.zeros((B, H), jnp.float32)
            hf_s[...] = z
            cf_s[...] = z
            hb_s[...] = z
            cb_s[...] = z

        xwf_s[...] = _dot(xf_ref[...].reshape(CH * B, D), wxf_ref[...])
        xwb_s[...] = _dot(xb_ref[...].reshape(CH * B, D), wxb_ref[...])

        hf, cf, hb, cb = hf_s[...], cf_s[...], hb_s[...], cb_s[...]
        whf = whf_ref[...]
        whb = whb_ref[...]
        bf = bf_ref[...]
        bb = bb_ref[...]
        for tl in range(CH):
            hf, cf = _lstm_step(xwf_s[tl * B:(tl + 1) * B, :], hf, cf, whf, bf)
            hf_o[:, tl, :] = hf
            tb = CH - 1 - tl
            hb, cb = _lstm_step(xwb_s[tb * B:(tb + 1) * B, :], hb, cb, whb, bb)
            hb_o[:, tb, :] = hb
        hf_s[...] = hf
        cf_s[...] = cf
        hb_s[...] = hb
        cb_s[...] = cb

    return pl.pallas_call(
        body,
        grid=(n_ch,),
        in_specs=[
            pl.BlockSpec((CH, B, D), lambda i: (i, 0, 0)),
            pl.BlockSpec((CH, B, D), lambda i: (n_ch - 1 - i, 0, 0)),
            pl.BlockSpec((D, 4 * H), lambda i: (0, 0)),
            pl.BlockSpec((D, 4 * H), lambda i: (0, 0)),
            pl.BlockSpec((H, 4 * H), lambda i: (0, 0)),
            pl.BlockSpec((H, 4 * H), lambda i: (0, 0)),
            pl.BlockSpec((1, 4 * H), lambda i: (0, 0)),
            pl.BlockSpec((1, 4 * H), lambda i: (0, 0)),
        ],
        out_specs=[
            pl.BlockSpec((B, CH, H), lambda i: (0, i, 0)),
            pl.BlockSpec((B, CH, H), lambda i: (0, n_ch - 1 - i, 0)),
        ],
        out_shape=[
            jax.ShapeDtypeStruct((B, T, H), jnp.float32),
            jax.ShapeDtypeStruct((B, T, H), jnp.float32),
        ],
        scratch_shapes=[pltpu.VMEM((CH * B, 4 * H), jnp.float32)] * 2
                       + [pltpu.VMEM((B, H), jnp.float32)] * 4,
        interpret=_INTERPRET,
    )(rows, rows, wx_f, wx_b, wh_f, wh_b, b_f, b_b)


def _attention(hpf, hpb, hqf, hqb, w_att, w_m, b_m2, w_se):
    def body(hpf_r, hpb_r, hqf_r, hqb_r, wa_r, wm_r, bm_r, wse_r, lo_r, pr_r):
        hp = jnp.concatenate([hpf_r[0], hpb_r[0]], axis=-1)
        hq = jnp.concatenate([hqf_r[0], hqb_r[0]], axis=-1)
        tmp = _dot_t(hq, wa_r[...])
        scores = _dot_t(hp, tmp)
        mx = jnp.max(scores, axis=-1, keepdims=True)
        e = jnp.exp(scores - mx)
        alpha = e / jnp.sum(e, axis=-1, keepdims=True)
        ctx = _dot(alpha, hq)
        cat = jnp.concatenate([hp, ctx], axis=-1)
        m = jnp.tanh(_dot(cat, wm_r[...]) + bm_r[...])
        lt = lax.dot_general(
            wse_r[...], m, (((0,), (1,)), ((), ())),
            preferred_element_type=jnp.float32,
        )
        lo_r[0] = lt
        iota = lax.broadcasted_iota(jnp.int32, (2, P), 1)
        mx2 = jnp.max(lt, axis=-1, keepdims=True)
        idx = jnp.min(jnp.where(lt == mx2, iota, P), axis=-1)
        pr_r[0] = idx.reshape(1, 2)

    return pl.pallas_call(
        body,
        grid=(B,),
        in_specs=[
            pl.BlockSpec((1, P, H), lambda b: (b, 0, 0)),
            pl.BlockSpec((1, P, H), lambda b: (b, 0, 0)),
            pl.BlockSpec((1, Q, H), lambda b: (b, 0, 0)),
            pl.BlockSpec((1, Q, H), lambda b: (b, 0, 0)),
            pl.BlockSpec((2 * H, 2 * H), lambda b: (0, 0)),
            pl.BlockSpec((4 * H, 2 * H), lambda b: (0, 0)),
            pl.BlockSpec((1, 2 * H), lambda b: (0, 0)),
            pl.BlockSpec((2 * H, 2), lambda b: (0, 0)),
        ],
        out_specs=[
            pl.BlockSpec((1, 2, P), lambda b: (b, 0, 0)),
            pl.BlockSpec((1, 1, 2), lambda b: (b, 0, 0)),
        ],
        out_shape=[
            jax.ShapeDtypeStruct((B, 2, P), jnp.float32),
            jax.ShapeDtypeStruct((B, 1, 2), jnp.int32),
        ],
        interpret=_INTERPRET,
    )(hpf, hpb, hqf, hqb, w_att, w_m, b_m2, w_se)


def kernel(passage, question, embedding,
           p_Wx_f, p_Wh_f, p_b_f, p_Wx_b, p_Wh_b, p_b_b,
           q_Wx_f, q_Wh_f, q_b_f, q_Wx_b, q_Wh_b, q_b_b,
           W_att, W_m, b_m, w_start, w_end):
    pidx = jnp.transpose(passage).reshape(-1).astype(jnp.int32)
    qidx = jnp.transpose(question[:, ::-1]).reshape(-1).astype(jnp.int32)
    idx = jnp.concatenate([pidx, qidx, jnp.zeros((Q_PAD - NQ_TOK,), jnp.int32)])

    rows = _gather_rows(embedding, idx)
    rows_p = rows[:NP_TOK].reshape(P, B, D)
    rows_q = rows[NP_TOK:NP_TOK + NQ_TOK].reshape(Q, B, D)

    hpf, hpb = _bilstm_scan(rows_p, p_Wx_f, p_Wx_b, p_Wh_f, p_Wh_b,
                            p_b_f.reshape(1, 4 * H), p_b_b.reshape(1, 4 * H), P, 40)
    hqf, hqb = _bilstm_scan(rows_q, q_Wx_f, q_Wx_b, q_Wh_f, q_Wh_b,
                            q_b_f.reshape(1, 4 * H), q_b_b.reshape(1, 4 * H), Q, 30)

    logits, preds = _attention(
        hpf, hpb, hqf, hqb, W_att, W_m,
        b_m.reshape(1, 2 * H), jnp.stack([w_start, w_end], axis=1),
    )
    return logits, preds.reshape(B, 2)
